# Initial kernel scaffold; baseline (speedup 1.0000x reference)
import jax
import jax.numpy as jnp
from jax.experimental import pallas as pl


def kernel(x, edge_index, edge_label_index, W1, b1, W2, b2):
    raise NotImplementedError("write your pallas kernel here")


# speedup vs baseline: 11.4067x; 11.4067x over previous
import functools

import jax
import jax.numpy as jnp
from jax import lax
from jax.experimental import pallas as pl
from jax.experimental.pallas import tpu as pltpu
from jax.experimental.pallas import tpu_sc as plsc

NC = 2
NS = 16
NW = NC * NS
CHUNK = 128


def _sc_mesh():
  return plsc.VectorSubcoreMesh(
      core_axis_name="c", subcore_axis_name="s",
      num_cores=NC, num_subcores=NS)


def _deg_sc(dst, ones2d, n):
  d = ones2d.shape[1]
  e = dst.shape[0]
  e_half = e // NC
  n_chunks = e_half // CHUNK
  iters = -(-n_chunks // NS)
  zrows = 8
  zchunks = n // zrows
  ziters = -(-zchunks // NS)
  crows = 40
  cchunks = n // crows
  citers = -(-cchunks // NS)

  @functools.partial(
      pl.kernel,
      out_type=jax.ShapeDtypeStruct((NC * n, d), jnp.float32),
      mesh=_sc_mesh(),
      scratch_types=[
          pltpu.VMEM((CHUNK,), jnp.int32),
          pltpu.VMEM((CHUNK, d), jnp.float32),
          pltpu.VMEM((zrows, d), jnp.float32),
          pltpu.VMEM_SHARED((n, d), jnp.float32),
      ],
  )
  def k(dst_hbm, ones_hbm, out_hbm, dst_v, ones_v, zero_v, acc_sh):
    cid = lax.axis_index("c")
    sid = lax.axis_index("s")
    zero16 = jnp.zeros((16,), jnp.float32)
    for r in range(zrows):
      for j in range(d // 16):
        zero_v[r, pl.ds(16 * j, 16)] = zero16
    pltpu.sync_copy(ones_hbm, ones_v)

    def zbody(i, carry):
      c = sid + NS * i

      @pl.when(c < zchunks)
      def _():
        pltpu.sync_copy(zero_v, acc_sh.at[pl.ds(c * zrows, zrows)])
      return carry

    lax.fori_loop(0, ziters, zbody, 0)
    plsc.subcore_barrier()

    def body(i, carry):
      c = sid + NS * i

      @pl.when(c < n_chunks)
      def _():
        off = cid * e_half + c * CHUNK
        pltpu.sync_copy(dst_hbm.at[pl.ds(off, CHUNK)], dst_v)
        pltpu.sync_copy(ones_v, acc_sh.at[dst_v], add=True)
      return carry

    lax.fori_loop(0, iters, body, 0)
    plsc.subcore_barrier()

    def obody(i, carry):
      c = sid + NS * i

      @pl.when(c < cchunks)
      def _():
        pltpu.sync_copy(acc_sh.at[pl.ds(c * crows, crows)],
                        out_hbm.at[pl.ds(cid * n + c * crows, crows)])
      return carry

    lax.fori_loop(0, citers, obody, 0)

  return k(dst, ones2d)


def _agg_sc(y, src, dst):
  n, d = y.shape
  e = src.shape[0]
  e_half = e // NC
  n_chunks = e_half // CHUNK
  iters = -(-n_chunks // NS)
  zrows = 8
  zchunks = n // zrows
  ziters = -(-zchunks // NS)
  crows = 40
  cchunks = n // crows
  citers = -(-cchunks // NS)

  @functools.partial(
      pl.kernel,
      out_type=jax.ShapeDtypeStruct((NC * n, d), jnp.float32),
      mesh=_sc_mesh(),
      scratch_types=[
          pltpu.VMEM((CHUNK,), jnp.int32),
          pltpu.VMEM((CHUNK,), jnp.int32),
          pltpu.VMEM((CHUNK, d), jnp.float32),
          pltpu.VMEM((zrows, d), jnp.float32),
          pltpu.VMEM_SHARED((n, d), jnp.float32),
          pltpu.SemaphoreType.DMA,
      ],
  )
  def k(y_hbm, src_hbm, dst_hbm, out_hbm, src_v, dst_v, rows_v, zero_v,
        acc_sh, sem):
    cid = lax.axis_index("c")
    sid = lax.axis_index("s")
    zero16 = jnp.zeros((16,), jnp.float32)
    for r in range(zrows):
      for j in range(d // 16):
        zero_v[r, pl.ds(16 * j, 16)] = zero16

    def zbody(i, carry):
      c = sid + NS * i

      @pl.when(c < zchunks)
      def _():
        pltpu.sync_copy(zero_v, acc_sh.at[pl.ds(c * zrows, zrows)])
      return carry

    lax.fori_loop(0, ziters, zbody, 0)
    plsc.subcore_barrier()

    def body(i, carry):
      c = sid + NS * i

      @pl.when(c < n_chunks)
      def _():
        off = cid * e_half + c * CHUNK
        pltpu.sync_copy(src_hbm.at[pl.ds(off, CHUNK)], src_v)
        pltpu.sync_copy(dst_hbm.at[pl.ds(off, CHUNK)], dst_v)
        pltpu.async_copy(y_hbm.at[src_v], rows_v, sem).wait()
        pltpu.sync_copy(rows_v, acc_sh.at[dst_v], add=True)
      return carry

    lax.fori_loop(0, iters, body, 0)
    plsc.subcore_barrier()

    def obody(i, carry):
      c = sid + NS * i

      @pl.when(c < cchunks)
      def _():
        pltpu.sync_copy(acc_sh.at[pl.ds(c * crows, crows)],
                        out_hbm.at[pl.ds(cid * n + c * crows, crows)])
      return carry

    lax.fori_loop(0, citers, obody, 0)

  return k(y, src, dst)


def _decode_sc(z, ai, bi):
  n, d = z.shape
  elp = ai.shape[0]
  per_tile = elp // NW
  kc = 112
  n_chunks = per_tile // kc
  nj = d // 16

  @functools.partial(
      pl.kernel,
      out_type=jax.ShapeDtypeStruct((elp, 16), jnp.float32),
      mesh=_sc_mesh(),
      scratch_types=[
          pltpu.VMEM((kc,), jnp.int32),
          pltpu.VMEM((kc,), jnp.int32),
          pltpu.VMEM((kc, d), jnp.float32),
          pltpu.VMEM((kc, d), jnp.float32),
          pltpu.VMEM((kc, 16), jnp.float32),
          pltpu.SemaphoreType.DMA,
          pltpu.SemaphoreType.DMA,
      ],
  )
  def k(z_hbm, a_hbm, b_hbm, out_hbm, ai_v, bi_v, za_v, zb_v, part_v, sa, sb):
    cid = lax.axis_index("c")
    sid = lax.axis_index("s")
    wid = sid * NC + cid
    base = wid * per_tile

    def body(ci, carry):
      off = base + ci * kc
      pltpu.sync_copy(a_hbm.at[pl.ds(off, kc)], ai_v)
      pltpu.sync_copy(b_hbm.at[pl.ds(off, kc)], bi_v)
      ca = pltpu.async_copy(z_hbm.at[ai_v], za_v, sa)
      cb = pltpu.async_copy(z_hbm.at[bi_v], zb_v, sb)
      ca.wait()
      cb.wait()

      def ebody(ei, ecarry):
        acc = za_v[ei, pl.ds(0, 16)] * zb_v[ei, pl.ds(0, 16)]
        for j in range(1, nj):
          acc = acc + za_v[ei, pl.ds(16 * j, 16)] * zb_v[ei, pl.ds(16 * j, 16)]
        part_v[ei, :] = acc
        return ecarry

      lax.fori_loop(0, kc, ebody, 0)
      pltpu.sync_copy(part_v, out_hbm.at[pl.ds(off, kc)])
      return carry

    lax.fori_loop(0, n_chunks, body, 0)

  return k(z, ai, bi)


def _tc_mm1(deg2, x, w1):
  n, d = x.shape
  rb = 1000
  g = n // rb

  def body(dega, degb, x_ref, w_ref, y_ref):
    deg = dega[:, 0:1] + degb[:, 0:1] + 1.0
    dinv = lax.rsqrt(deg)
    y_ref[...] = dinv * jnp.dot(x_ref[...], w_ref[...],
                                preferred_element_type=jnp.float32)

  return pl.pallas_call(
      body,
      grid=(g,),
      in_specs=[
          pl.BlockSpec((rb, d), lambda i: (i, 0)),
          pl.BlockSpec((rb, d), lambda i: (i + g, 0)),
          pl.BlockSpec((rb, d), lambda i: (i, 0)),
          pl.BlockSpec((d, d), lambda i: (0, 0)),
      ],
      out_specs=pl.BlockSpec((rb, d), lambda i: (i, 0)),
      out_shape=jax.ShapeDtypeStruct((n, d), jnp.float32),
  )(deg2, deg2, x, w1)


def _tc_mm2(deg2, acc2, y1, b1, w2):
  n, d = y1.shape
  rb = 1000
  g = n // rb

  def body(dega, degb, acca, accb, y_ref, b_ref, w_ref, out_ref):
    deg = dega[:, 0:1] + degb[:, 0:1] + 1.0
    dinv = lax.rsqrt(deg)
    s = (acca[...] + accb[...] + y_ref[...]) * dinv + b_ref[...]
    h = jnp.maximum(s, 0.0)
    out_ref[...] = dinv * jnp.dot(h, w_ref[...],
                                  preferred_element_type=jnp.float32)

  return pl.pallas_call(
      body,
      grid=(g,),
      in_specs=[
          pl.BlockSpec((rb, d), lambda i: (i, 0)),
          pl.BlockSpec((rb, d), lambda i: (i + g, 0)),
          pl.BlockSpec((rb, d), lambda i: (i, 0)),
          pl.BlockSpec((rb, d), lambda i: (i + g, 0)),
          pl.BlockSpec((rb, d), lambda i: (i, 0)),
          pl.BlockSpec((1, d), lambda i: (0, 0)),
          pl.BlockSpec((d, d), lambda i: (0, 0)),
      ],
      out_specs=pl.BlockSpec((rb, d), lambda i: (i, 0)),
      out_shape=jax.ShapeDtypeStruct((n, d), jnp.float32),
  )(deg2, deg2, acc2, acc2, y1, b1, w2)


def _tc_fin(deg2, acc2, y2, b2):
  n, d = y2.shape
  rb = 1000
  g = n // rb

  def body(dega, degb, acca, accb, y_ref, b_ref, out_ref):
    deg = dega[:, 0:1] + degb[:, 0:1] + 1.0
    dinv = lax.rsqrt(deg)
    out_ref[...] = (acca[...] + accb[...] + y_ref[...]) * dinv + b_ref[...]

  return pl.pallas_call(
      body,
      grid=(g,),
      in_specs=[
          pl.BlockSpec((rb, d), lambda i: (i, 0)),
          pl.BlockSpec((rb, d), lambda i: (i + g, 0)),
          pl.BlockSpec((rb, d), lambda i: (i, 0)),
          pl.BlockSpec((rb, d), lambda i: (i + g, 0)),
          pl.BlockSpec((rb, d), lambda i: (i, 0)),
          pl.BlockSpec((1, d), lambda i: (0, 0)),
      ],
      out_specs=pl.BlockSpec((rb, d), lambda i: (i, 0)),
      out_shape=jax.ShapeDtypeStruct((n, d), jnp.float32),
  )(deg2, deg2, acc2, acc2, y2, b2)


def _tc_lanesum(part):
  elp = part.shape[0]
  g = 32
  rb = elp // g

  def body(p_ref, out_ref):
    out_ref[...] = jnp.sum(p_ref[...], axis=1, keepdims=True)

  return pl.pallas_call(
      body,
      grid=(g,),
      in_specs=[pl.BlockSpec((rb, 16), lambda i: (i, 0))],
      out_specs=pl.BlockSpec((rb, 1), lambda i: (i, 0)),
      out_shape=jax.ShapeDtypeStruct((elp, 1), jnp.float32),
  )(part)


def kernel(x, edge_index, edge_label_index, W1, b1, W2, b2):
  n, d = x.shape
  el = edge_label_index.shape[1]
  src = edge_index[0].astype(jnp.int32)
  dst = edge_index[1].astype(jnp.int32)

  kc = 112
  per_tile = -(-el // NW)
  per_tile = -(-per_tile // kc) * kc
  elp = per_tile * NW
  pad = elp - el
  ai = jnp.concatenate(
      [edge_label_index[0].astype(jnp.int32), jnp.zeros((pad,), jnp.int32)])
  bi = jnp.concatenate(
      [edge_label_index[1].astype(jnp.int32), jnp.zeros((pad,), jnp.int32)])

  deg2 = _deg_sc(dst, jnp.ones((CHUNK, d), jnp.float32), n)
  y1 = _tc_mm1(deg2, x, W1)
  acc1 = _agg_sc(y1, src, dst)
  y2 = _tc_mm2(deg2, acc1, y1, b1.reshape(1, d), W2)
  acc2 = _agg_sc(y2, src, dst)
  z = _tc_fin(deg2, acc2, y2, b2.reshape(1, d))
  part = _decode_sc(z, ai, bi)
  logits = _tc_lanesum(part)
  return logits[:el, 0]
